# Initial kernel scaffold; baseline (speedup 1.0000x reference)
import jax
import jax.numpy as jnp
from jax.experimental import pallas as pl


def kernel(x, I):
    raise NotImplementedError("write your pallas kernel here")


# speedup vs baseline: 29.3936x; 29.3936x over previous
import functools

import jax
import jax.numpy as jnp
from jax import lax
from jax.experimental import pallas as pl
from jax.experimental.pallas import tpu as pltpu
from jax.experimental.pallas import tpu_sc as plsc

B, C, G, S, L = 32, 16, 2048, 8, 3
NC, NS = 2, 16
NW = NC * NS
GW = G // NW
NGG = GW // 16
IDX_W = C * S * L * GW
OUT_W = C * B * GW


def _sc_body(x_hbm, iw_hbm, out_hbm, x_v, idx_v, out_v):
    wid = lax.axis_index("s") * NC + lax.axis_index("c")
    pltpu.sync_copy(x_hbm, x_v)
    pltpu.sync_copy(iw_hbm.at[wid], idx_v)

    def t_body(t, _):
        c = t // NGG
        gg = t - c * NGG
        base = c * (S * L * GW) + gg * 16
        iv = [[idx_v[pl.ds(base + (s * L + l) * GW, 16)]
               for l in range(L)] for s in range(S)]
        obase = c * (B * GW) + gg * 16

        def b_body(b, _):
            boff = b * G
            acc = None
            for s in range(S):
                p = plsc.load_gather(x_v, [iv[s][0] + boff])
                p = p * plsc.load_gather(x_v, [iv[s][1] + boff])
                p = p * plsc.load_gather(x_v, [iv[s][2] + boff])
                acc = p if acc is None else acc + p
            out_v[pl.ds(obase + b * GW, 16)] = acc
            return 0

        lax.fori_loop(0, B, b_body, 0)
        return 0

    lax.fori_loop(0, C * NGG, t_body, 0)
    pltpu.sync_copy(out_v, out_hbm.at[wid])


_sc_call = functools.partial(
    pl.kernel,
    out_type=jax.ShapeDtypeStruct((NW, OUT_W), jnp.float32),
    mesh=plsc.VectorSubcoreMesh(core_axis_name="c", subcore_axis_name="s"),
    compiler_params=pltpu.CompilerParams(needs_layout_passes=False),
    scratch_types=[
        pltpu.VMEM((B * G,), jnp.float32),
        pltpu.VMEM((IDX_W,), jnp.int32),
        pltpu.VMEM((OUT_W,), jnp.float32),
    ],
)(_sc_body)


def kernel(x, I):
    I_w = (I.transpose(0, 2, 3, 1)
            .reshape(C, S, L, NW, GW)
            .transpose(3, 0, 1, 2, 4)
            .reshape(NW, IDX_W))
    out_w = _sc_call(x.reshape(-1), I_w)
    return (out_w.reshape(NW, C, B, GW)
                 .transpose(1, 2, 0, 3)
                 .reshape(C, B, G))
